# baseline (speedup 1.0000x reference)
import functools

import jax
import jax.numpy as jnp
from jax import lax
from jax.experimental import pallas as pl
from jax.experimental.pallas import tpu as pltpu
from jax.experimental.pallas import tpu_sc as plsc

EMBED = 64
SCALE = 8.0

NC = 2
NS = 16
NW = NC * NS

CHUNK = 128
TOKENS_TOTAL = 4096 * 200
B_PER_W = TOKENS_TOTAL // NW
NCH = B_PER_W // CHUNK
NBUF = 2


@functools.partial(
    pl.kernel,
    mesh=plsc.VectorSubcoreMesh(core_axis_name="c", subcore_axis_name="s"),
    out_type=jax.ShapeDtypeStruct((NW, NCH, CHUNK, EMBED), jnp.float32),
    scratch_types=[
        pltpu.VMEM((NCH, CHUNK), jnp.int32),
        pltpu.VMEM((NBUF, CHUNK, EMBED), jnp.float32),
        pltpu.SemaphoreType.DMA,
        pltpu.SemaphoreType.DMA,
    ],
    compiler_params=pltpu.CompilerParams(use_tc_tiling_on_sc=False),
)
def _embed_gather(idx_hbm, table_hbm, out_hbm, idx_v, rows_v, ld_sem, st_sem):
    wid = lax.axis_index("s") * NC + lax.axis_index("c")

    pltpu.sync_copy(idx_hbm.at[wid], idx_v)

    def gather(j, buf):
        return pltpu.async_copy(
            table_hbm.at[idx_v.at[j]], rows_v.at[buf], ld_sem)

    def scale(buf):
        def row(r, _):
            for q in range(EMBED // 16):
                sl = pl.ds(q * 16, 16)
                rows_v[buf, r, sl] = rows_v[buf, r, sl] * SCALE
            return 0
        lax.fori_loop(0, CHUNK, row, 0, unroll=2)

    gather(0, 0)

    def chunk_body(j, _):
        buf = lax.rem(j, NBUF)

        @pl.when(j + 1 < NCH)
        def _():
            gather(j + 1, lax.rem(j + 1, NBUF))

        pltpu.make_async_copy(
            table_hbm.at[idx_v.at[j]], rows_v.at[buf], ld_sem).wait()
        scale(buf)
        pltpu.async_copy(rows_v.at[buf], out_hbm.at[wid, j], st_sem).wait()
        return 0

    lax.fori_loop(0, NCH, chunk_body, 0)


def kernel(tokens, table):
    idx = tokens.astype(jnp.int32).reshape(NW, NCH, CHUNK)
    out = _embed_gather(idx, table)
    return out.reshape(tokens.shape[0], tokens.shape[1], EMBED)


# speedup vs baseline: 1.0013x; 1.0013x over previous
import functools

import jax
import jax.numpy as jnp
from jax import lax
from jax.experimental import pallas as pl
from jax.experimental.pallas import tpu as pltpu
from jax.experimental.pallas import tpu_sc as plsc

EMBED = 64
SCALE = 8.0

NC = 2
NS = 16
NW = NC * NS

CHUNK = 128
TOKENS_TOTAL = 4096 * 200
B_PER_W = TOKENS_TOTAL // NW
NCH = B_PER_W // CHUNK
NBUF = 2


@functools.partial(
    pl.kernel,
    mesh=plsc.VectorSubcoreMesh(core_axis_name="c", subcore_axis_name="s"),
    out_type=jax.ShapeDtypeStruct((NW, NCH, CHUNK, EMBED), jnp.float32),
    scratch_types=[
        pltpu.VMEM((NCH, CHUNK), jnp.int32),
        pltpu.VMEM((NBUF, CHUNK, EMBED), jnp.float32),
        pltpu.SemaphoreType.DMA,
        pltpu.SemaphoreType.DMA,
    ],
    compiler_params=pltpu.CompilerParams(
        use_tc_tiling_on_sc=False, skip_device_barrier=True),
)
def _embed_gather(idx_hbm, table_hbm, out_hbm, idx_v, rows_v, ld_sem, st_sem):
    wid = lax.axis_index("s") * NC + lax.axis_index("c")

    pltpu.sync_copy(idx_hbm.at[wid], idx_v)

    def gather(j, buf):
        return pltpu.async_copy(
            table_hbm.at[idx_v.at[j]], rows_v.at[buf], ld_sem)

    def scale(buf):
        def row(r, _):
            for q in range(EMBED // 16):
                sl = pl.ds(q * 16, 16)
                rows_v[buf, r, sl] = rows_v[buf, r, sl] * SCALE
            return 0
        lax.fori_loop(0, CHUNK, row, 0, unroll=2)

    gather(0, 0)

    def chunk_body(j, _):
        buf = lax.rem(j, NBUF)

        @pl.when(j + 1 < NCH)
        def _():
            gather(j + 1, lax.rem(j + 1, NBUF))

        pltpu.make_async_copy(
            table_hbm.at[idx_v.at[j]], rows_v.at[buf], ld_sem).wait()
        scale(buf)
        pltpu.async_copy(rows_v.at[buf], out_hbm.at[wid, j], st_sem).wait()
        return 0

    lax.fori_loop(0, NCH, chunk_body, 0)


def kernel(tokens, table):
    idx = tokens.astype(jnp.int32).reshape(NW, NCH, CHUNK)
    out = _embed_gather(idx, table)
    return out.reshape(tokens.shape[0], tokens.shape[1], EMBED)


# speedup vs baseline: 1.0016x; 1.0003x over previous
import functools

import jax
import jax.numpy as jnp
from jax import lax
from jax.experimental import pallas as pl
from jax.experimental.pallas import tpu as pltpu
from jax.experimental.pallas import tpu_sc as plsc

EMBED = 64
SCALE = 8.0

NC = 2
NS = 16
NW = NC * NS

ROWS = 4096
COLS = 200
R_PER_W = ROWS // NW
GCHUNK = 40
NG = COLS // GCHUNK


@functools.partial(
    pl.kernel,
    mesh=plsc.VectorSubcoreMesh(core_axis_name="c", subcore_axis_name="s"),
    out_type=jax.ShapeDtypeStruct((ROWS, COLS, EMBED), jnp.float32),
    scratch_types=[
        pltpu.VMEM((R_PER_W, COLS), jnp.int32),
        pltpu.VMEM((COLS, EMBED), jnp.float32),
        pltpu.VMEM((COLS, EMBED), jnp.float32),
        pltpu.SemaphoreType.DMA,
        pltpu.SemaphoreType.DMA,
    ],
    compiler_params=pltpu.CompilerParams(use_tc_tiling_on_sc=False),
)
def _embed_gather(idx_hbm, table_hbm, out_hbm, idx_v, row0_v, row1_v,
                  sem0, sem1):
    wid = lax.axis_index("s") * NC + lax.axis_index("c")
    base = wid * R_PER_W

    pltpu.sync_copy(idx_hbm.at[pl.ds(base, R_PER_W)], idx_v)

    def issue(r, buf, sem):
        for k in range(NG):
            sl = pl.ds(k * GCHUNK, GCHUNK)
            pltpu.async_copy(
                table_hbm.at[idx_v.at[r, sl]], buf.at[sl], sem)

    def drain(buf, sem):
        pltpu.make_async_copy(table_hbm.at[idx_v.at[0]], buf, sem).wait()

    def scale(buf):
        def row(r, _):
            for q in range(EMBED // 16):
                sl = pl.ds(q * 16, 16)
                buf[r, sl] = buf[r, sl] * SCALE
            return 0
        lax.fori_loop(0, COLS, row, 0)

    issue(0, row0_v, sem0)

    def pair_body(t, _):
        r0 = 2 * t
        issue(r0 + 1, row1_v, sem1)
        drain(row0_v, sem0)
        scale(row0_v)
        pltpu.sync_copy(row0_v, out_hbm.at[base + r0])

        @pl.when(r0 + 2 < R_PER_W)
        def _():
            issue(r0 + 2, row0_v, sem0)

        drain(row1_v, sem1)
        scale(row1_v)
        pltpu.sync_copy(row1_v, out_hbm.at[base + r0 + 1])
        return 0

    lax.fori_loop(0, R_PER_W // 2, pair_body, 0)


def kernel(tokens, table):
    return _embed_gather(tokens.astype(jnp.int32), table)
